# baseline (speedup 1.0000x reference)
import functools

import jax
import jax.numpy as jnp
from jax import lax
from jax.experimental import pallas as pl
from jax.experimental.pallas import tpu as pltpu
from jax.experimental.pallas import tpu_sc as plsc

B = 128
N = 512
F = 64
PE = 500
GE = 5000
HID = 128
GED = 128
CLIN = 6
NCLS = 2

BP = 8
NWORK = 32
BH = B // 2
PPW = BH // NWORK
KBLK = N // 128
PLANE = KBLK * N * 128
ZCH = 65536
PEP = 512
GEP = 5120


def _sc_build_body(with_group, pe_hbm, ge_hbm, outp_hbm, outg_hbm,
                   zbuf, ebuf, ibuf0, ibuf1, vbuf, sem, sem1, sem2, sem3):
    ibufs = (ibuf0, ibuf1)
    sems = (sem, sem1, sem2, sem3)
    wid = lax.axis_index("s") * 2 + lax.axis_index("c")

    def _fill_z(i, _):
        zbuf[pl.ds(i * 16, 16)] = jnp.zeros((16,), jnp.float32)
        return 0

    def _fill_o(i, _):
        vbuf[pl.ds(i * 16, 16)] = jnp.ones((16,), jnp.float32)
        return 0

    lax.fori_loop(0, ZCH // 16, _fill_z, 0, unroll=8)
    lax.fori_loop(0, PEP // 16, _fill_o, 0, unroll=8)

    base = wid * (PPW * PLANE)
    copies = [
        pltpu.async_copy(zbuf, outp_hbm.at[pl.ds(base + i * ZCH, ZCH)],
                         sems[i % len(sems)])
        for i in range(PPW * PLANE // ZCH)
    ]
    for cp in copies:
        cp.wait()
    if with_group:
        @pl.when(wid == 0)
        def _():
            pltpu.sync_copy(zbuf.at[pl.ds(0, B * B)], outg_hbm)

    scatters = []
    for j in range(PPW):
        b = wid * PPW + j
        pltpu.sync_copy(pe_hbm.at[b], ebuf.at[j])
        pbase = b * PLANE

        def _pidx(c, _, j=j, pbase=pbase):
            s = ebuf[j, 0, pl.ds(c * 16, 16)]
            d = ebuf[j, 1, pl.ds(c * 16, 16)]
            flat = (pbase + lax.shift_right_logical(s, 7) * (N * 128)
                    + d * 128 + lax.bitwise_and(s, 127))
            ibufs[j][pl.ds(c * 16, 16)] = flat
            return 0

        lax.fori_loop(0, PEP // 16, _pidx, 0, unroll=4)
        scatters.append(
            pltpu.async_copy(vbuf, outp_hbm.at[ibufs[j]], sem))
    for cp in scatters:
        cp.wait()

    if with_group:
        @pl.when(wid == 0)
        def _():
            for g in range(GEP // PEP):
                pltpu.sync_copy(ge_hbm.at[0, pl.ds(g * PEP, PEP)],
                                ebuf.at[0, 0])
                pltpu.sync_copy(ge_hbm.at[1, pl.ds(g * PEP, PEP)],
                                ebuf.at[0, 1])

                def _gidx(c, _):
                    s = ebuf[0, 0, pl.ds(c * 16, 16)]
                    d = ebuf[0, 1, pl.ds(c * 16, 16)]
                    ibuf0[pl.ds(c * 16, 16)] = d * B + s
                    return 0

                lax.fori_loop(0, PEP // 16, _gidx, 0, unroll=4)
                pltpu.async_copy(vbuf, outg_hbm.at[ibuf0], sem).wait()


_SC_SCRATCH = [
    pltpu.VMEM((ZCH,), jnp.float32),
    pltpu.VMEM((PPW, 2, PEP), jnp.int32),
    pltpu.VMEM((PEP,), jnp.int32),
    pltpu.VMEM((PEP,), jnp.int32),
    pltpu.VMEM((PEP,), jnp.float32),
    pltpu.SemaphoreType.DMA,
    pltpu.SemaphoreType.DMA,
    pltpu.SemaphoreType.DMA,
    pltpu.SemaphoreType.DMA,
]
_SC_MESH = plsc.VectorSubcoreMesh(core_axis_name="c", subcore_axis_name="s")


@functools.partial(
    pl.kernel,
    out_type=jax.ShapeDtypeStruct((BH * PLANE,), jnp.float32),
    mesh=_SC_MESH, scratch_types=list(_SC_SCRATCH),
)
def _sc_build_half(pe_hbm, outp_hbm, zbuf, ebuf, ibuf0, ibuf1, vbuf,
                   sem, sem1, sem2, sem3):
    _sc_build_body(False, pe_hbm, None, outp_hbm, None,
                   zbuf, ebuf, ibuf0, ibuf1, vbuf, sem, sem1, sem2, sem3)


@functools.partial(
    pl.kernel,
    out_type=(jax.ShapeDtypeStruct((BH * PLANE,), jnp.float32),
              jax.ShapeDtypeStruct((B * B,), jnp.float32)),
    mesh=_SC_MESH, scratch_types=list(_SC_SCRATCH),
)
def _sc_build_half_g(pe_hbm, ge_hbm, outp_hbm, outg_hbm,
                     zbuf, ebuf, ibuf0, ibuf1, vbuf, sem, sem1, sem2, sem3):
    _sc_build_body(True, pe_hbm, ge_hbm, outp_hbm, outg_hbm,
                   zbuf, ebuf, ibuf0, ibuf1, vbuf, sem, sem1, sem2, sem3)


def _patient_body(a_ref, x_ref, w1_ref, b1_ref, w2_ref, b2_ref,
                  w3_ref, b3_ref, plw_ref, plb_ref, out_ref):
    w1 = w1_ref[...]
    w2 = w2_ref[...]
    w3 = w3_ref[...]
    b1 = b1_ref[...]
    b2 = b2_ref[...]
    b3 = b3_ref[...]
    for p in range(BP):
        abar = jnp.concatenate([a_ref[p, k] for k in range(KBLK)], axis=1)
        deg = jnp.sum(abar, axis=1, keepdims=True) + 1.0
        dis = lax.rsqrt(deg)
        dis2 = dis * dis
        h = x_ref[p]
        for w, bb in ((w1, b1), (w2, b2), (w3, b3)):
            z = jnp.dot(h, w, preferred_element_type=jnp.float32)
            zn = dis * z
            acc = jnp.dot(abar, zn, preferred_element_type=jnp.float32)
            h = jnp.maximum(dis * acc + dis2 * z + bb, 0.0)
        g = jnp.max(h, axis=0, keepdims=True)
        out_ref[p:p + 1, :] = (
            jnp.dot(g, plw_ref[...], preferred_element_type=jnp.float32)
            + plb_ref[...])


def _group_body(ag_ref, emb_ref, demo_ref, w1a_ref, w1b_ref, b1_ref,
                w2_ref, b2_ref, w3_ref, b3_ref, w4_ref, b4_ref, out_ref):
    abar = ag_ref[...]
    deg = jnp.sum(abar, axis=1, keepdims=True) + 1.0
    dis = lax.rsqrt(deg)
    dis2 = dis * dis
    an = dis * abar * jnp.transpose(dis)

    z = (jnp.dot(emb_ref[...], w1a_ref[...], preferred_element_type=jnp.float32)
         + jnp.dot(demo_ref[...], w1b_ref[...], preferred_element_type=jnp.float32))
    h = jnp.maximum(jnp.dot(an, z, preferred_element_type=jnp.float32)
                    + dis2 * z + b1_ref[...], 0.0)
    for w_ref, b_ref, act in ((w2_ref, b2_ref, True), (w3_ref, b3_ref, True),
                              (w4_ref, b4_ref, False)):
        z = jnp.dot(h, w_ref[...], preferred_element_type=jnp.float32)
        h = (jnp.dot(an, z, preferred_element_type=jnp.float32)
             + dis2 * z + b_ref[...])
        if act:
            h = jnp.maximum(h, 0.0)
    m = jnp.max(h, axis=1, keepdims=True)
    y = h - m
    out_ref[...] = y - jnp.log(jnp.sum(jnp.exp(y), axis=1, keepdims=True))


def kernel(x, demographic, patient_edge_idx, group_edge_idx,
           pW1, pb1, pW2, pb2, pW3, pb3, plinW, plinb,
           gW1, gb1, gW2, gb2, gW3, gb3, gW4, gb4):
    pe_pad = jnp.concatenate(
        [patient_edge_idx,
         jnp.tile(patient_edge_idx[:, :, -1:], (1, 1, PEP - PE))], axis=2)
    ge_pad = jnp.concatenate(
        [group_edge_idx,
         jnp.tile(group_edge_idx[:, -1:], (1, GEP - GE))], axis=1)

    abar1_flat, ag_flat = _sc_build_half_g(pe_pad[:BH], ge_pad)
    abar2_flat = _sc_build_half(pe_pad[BH:])
    ag = ag_flat.reshape(B, B)

    row = lambda v: v.reshape(1, -1)
    fullg = lambda a: pl.BlockSpec(a.shape, lambda i: (0,) * a.ndim)
    full = lambda a: pl.BlockSpec(a.shape, lambda: (0,) * a.ndim)

    wargs = (pW1, row(pb1), pW2, row(pb2), pW3, row(pb3), plinW, row(plinb))
    patient_call = pl.pallas_call(
        _patient_body,
        grid=(BH // BP,),
        in_specs=[
            pl.BlockSpec((BP, KBLK, N, 128), lambda i: (i, 0, 0, 0)),
            pl.BlockSpec((BP, N, F), lambda i: (i, 0, 0)),
        ] + [fullg(a) for a in wargs],
        out_specs=pl.BlockSpec((BP, GED), lambda i: (i, 0)),
        out_shape=jax.ShapeDtypeStruct((BH, GED), jnp.float32),
    )
    embed1 = patient_call(abar1_flat.reshape(BH, KBLK, N, 128), x[:BH], *wargs)
    embed2 = patient_call(abar2_flat.reshape(BH, KBLK, N, 128), x[BH:], *wargs)
    embed = jnp.concatenate([embed1, embed2], axis=0)

    gw1a = gW1[:GED]
    gw1b = gW1[GED:]
    gargs = (ag, embed, demographic, gw1a, gw1b, row(gb1),
             gW2, row(gb2), gW3, row(gb3), gW4, row(gb4))
    out = pl.pallas_call(
        _group_body,
        in_specs=[full(a) for a in gargs],
        out_specs=pl.BlockSpec((B, NCLS), lambda: (0, 0)),
        out_shape=jax.ShapeDtypeStruct((B, NCLS), jnp.float32),
    )(*gargs)
    return out


# speedup vs baseline: 1.0317x; 1.0317x over previous
import functools

import jax
import jax.numpy as jnp
from jax import lax
from jax.experimental import pallas as pl
from jax.experimental.pallas import tpu as pltpu
from jax.experimental.pallas import tpu_sc as plsc

B = 128
N = 512
F = 64
PE = 500
GE = 5000
HID = 128
GED = 128
CLIN = 6
NCLS = 2

BP = 8
NWORK = 32
PPW = B // NWORK
NWRD = N // 16
BMP = N * NWRD
PEP = 512
GEP = 5120


def _sc_build_body(pe_hbm, ge_hbm, bm_hbm, outg_hbm,
                   zbuf, wbuf, ebuf, ibuf, vbuf, sem):
    wid = lax.axis_index("s") * 2 + lax.axis_index("c")
    lane = lax.broadcasted_iota(jnp.int32, (16,), 0)

    @pl.when(wid == 0)
    def _():
        def _fill_z(i, _):
            zbuf[pl.ds(i * 16, 16)] = jnp.zeros((16,), jnp.float32)
            return 0

        def _fill_o(i, _):
            vbuf[pl.ds(i * 16, 16)] = jnp.ones((16,), jnp.float32)
            return 0

        lax.fori_loop(0, B * B // 16, _fill_z, 0, unroll=8)
        lax.fori_loop(0, PEP // 16, _fill_o, 0, unroll=8)
        pltpu.sync_copy(zbuf, outg_hbm)

    for j in range(PPW):
        b = wid * PPW + j

        def _zero_w(i, _):
            wbuf[pl.ds(i * 16, 16)] = jnp.zeros((16,), jnp.int32)
            return 0

        lax.fori_loop(0, BMP // 16, _zero_w, 0, unroll=8)
        pltpu.sync_copy(pe_hbm.at[b], ebuf)

        def _edges(c, _):
            s = ebuf[0, pl.ds(c * 16, 16)]
            d = ebuf[1, pl.ds(c * 16, 16)]
            widx = d * NWRD + lax.shift_right_logical(s, 4)
            bit = lax.shift_left(jnp.full((16,), 1, jnp.int32),
                                 lax.bitwise_and(s, 15))
            mask = (c * 16 + lane) < PE
            plsc.addupdate_scatter(wbuf, [widx], bit, mask=mask)
            return 0

        lax.fori_loop(0, PEP // 16, _edges, 0, unroll=4)
        pltpu.sync_copy(wbuf, bm_hbm.at[pl.ds(b * BMP, BMP)])

    @pl.when(wid == 0)
    def _():
        for g in range(GEP // PEP):
            pltpu.sync_copy(ge_hbm.at[0, pl.ds(g * PEP, PEP)], ebuf.at[0])
            pltpu.sync_copy(ge_hbm.at[1, pl.ds(g * PEP, PEP)], ebuf.at[1])

            def _gidx(c, _):
                s = ebuf[0, pl.ds(c * 16, 16)]
                d = ebuf[1, pl.ds(c * 16, 16)]
                ibuf[pl.ds(c * 16, 16)] = d * B + s
                return 0

            lax.fori_loop(0, PEP // 16, _gidx, 0, unroll=4)
            pltpu.async_copy(vbuf, outg_hbm.at[ibuf], sem).wait()


@functools.partial(
    pl.kernel,
    out_type=(jax.ShapeDtypeStruct((B * BMP,), jnp.int32),
              jax.ShapeDtypeStruct((B * B,), jnp.float32)),
    mesh=plsc.VectorSubcoreMesh(core_axis_name="c", subcore_axis_name="s"),
    scratch_types=[
        pltpu.VMEM((B * B,), jnp.float32),
        pltpu.VMEM((BMP,), jnp.int32),
        pltpu.VMEM((2, PEP), jnp.int32),
        pltpu.VMEM((PEP,), jnp.int32),
        pltpu.VMEM((PEP,), jnp.float32),
        pltpu.SemaphoreType.DMA,
    ],
    compiler_params=pltpu.CompilerParams(needs_layout_passes=False),
)
def _sc_build(pe_hbm, ge_hbm, bm_hbm, outg_hbm,
              zbuf, wbuf, ebuf, ibuf, vbuf, sem):
    _sc_build_body(pe_hbm, ge_hbm, bm_hbm, outg_hbm,
                   zbuf, wbuf, ebuf, ibuf, vbuf, sem)


def _patient_body(a_ref, x_ref, w1_ref, b1_ref, w2_ref, b2_ref,
                  w3_ref, b3_ref, plw_ref, plb_ref, out_ref):
    w1 = w1_ref[...]
    w2 = w2_ref[...]
    w3 = w3_ref[...]
    b1 = b1_ref[...]
    b2 = b2_ref[...]
    b3 = b3_ref[...]
    sel = (lax.broadcasted_iota(jnp.int32, (NWRD, N), 1) // 16
           == lax.broadcasted_iota(jnp.int32, (NWRD, N), 0)
           ).astype(jnp.float32)
    shift = lax.broadcasted_iota(jnp.int32, (N, N), 1) & 15
    for p in range(BP):
        bmf = a_ref[p].astype(jnp.float32)
        v = jnp.dot(bmf, sel, preferred_element_type=jnp.float32)
        abar = ((v.astype(jnp.int32) >> shift) & 1).astype(jnp.float32)
        deg = jnp.sum(abar, axis=1, keepdims=True) + 1.0
        dis = lax.rsqrt(deg)
        dis2 = dis * dis
        h = x_ref[p]
        for w, bb in ((w1, b1), (w2, b2), (w3, b3)):
            z = jnp.dot(h, w, preferred_element_type=jnp.float32)
            zn = dis * z
            acc = jnp.dot(abar, zn, preferred_element_type=jnp.float32)
            h = jnp.maximum(dis * acc + dis2 * z + bb, 0.0)
        g = jnp.max(h, axis=0, keepdims=True)
        out_ref[p:p + 1, :] = (
            jnp.dot(g, plw_ref[...], preferred_element_type=jnp.float32)
            + plb_ref[...])


def _group_body(ag_ref, emb_ref, demo_ref, w1a_ref, w1b_ref, b1_ref,
                w2_ref, b2_ref, w3_ref, b3_ref, w4_ref, b4_ref, out_ref):
    abar = ag_ref[...]
    deg = jnp.sum(abar, axis=1, keepdims=True) + 1.0
    dis = lax.rsqrt(deg)
    dis2 = dis * dis
    an = dis * abar * jnp.transpose(dis)

    z = (jnp.dot(emb_ref[...], w1a_ref[...], preferred_element_type=jnp.float32)
         + jnp.dot(demo_ref[...], w1b_ref[...], preferred_element_type=jnp.float32))
    h = jnp.maximum(jnp.dot(an, z, preferred_element_type=jnp.float32)
                    + dis2 * z + b1_ref[...], 0.0)
    for w_ref, b_ref, act in ((w2_ref, b2_ref, True), (w3_ref, b3_ref, True),
                              (w4_ref, b4_ref, False)):
        z = jnp.dot(h, w_ref[...], preferred_element_type=jnp.float32)
        h = (jnp.dot(an, z, preferred_element_type=jnp.float32)
             + dis2 * z + b_ref[...])
        if act:
            h = jnp.maximum(h, 0.0)
    m = jnp.max(h, axis=1, keepdims=True)
    y = h - m
    out_ref[...] = y - jnp.log(jnp.sum(jnp.exp(y), axis=1, keepdims=True))


def kernel(x, demographic, patient_edge_idx, group_edge_idx,
           pW1, pb1, pW2, pb2, pW3, pb3, plinW, plinb,
           gW1, gb1, gW2, gb2, gW3, gb3, gW4, gb4):
    pe_pad = jnp.concatenate(
        [patient_edge_idx,
         jnp.tile(patient_edge_idx[:, :, -1:], (1, 1, PEP - PE))], axis=2)
    ge_pad = jnp.concatenate(
        [group_edge_idx,
         jnp.tile(group_edge_idx[:, -1:], (1, GEP - GE))], axis=1)

    bm_flat, ag_flat = _sc_build(pe_pad, ge_pad)
    bm = bm_flat.reshape(B, N, NWRD)
    ag = ag_flat.reshape(B, B)

    row = lambda v: v.reshape(1, -1)
    fullg = lambda a: pl.BlockSpec(a.shape, lambda i: (0,) * a.ndim)
    full = lambda a: pl.BlockSpec(a.shape, lambda: (0,) * a.ndim)

    wargs = (pW1, row(pb1), pW2, row(pb2), pW3, row(pb3), plinW, row(plinb))
    embed = pl.pallas_call(
        _patient_body,
        grid=(B // BP,),
        in_specs=[
            pl.BlockSpec((BP, N, NWRD), lambda i: (i, 0, 0)),
            pl.BlockSpec((BP, N, F), lambda i: (i, 0, 0)),
        ] + [fullg(a) for a in wargs],
        out_specs=pl.BlockSpec((BP, GED), lambda i: (i, 0)),
        out_shape=jax.ShapeDtypeStruct((B, GED), jnp.float32),
    )(bm, x, *wargs)

    gw1a = gW1[:GED]
    gw1b = gW1[GED:]
    gargs = (ag, embed, demographic, gw1a, gw1b, row(gb1),
             gW2, row(gb2), gW3, row(gb3), gW4, row(gb4))
    out = pl.pallas_call(
        _group_body,
        in_specs=[full(a) for a in gargs],
        out_specs=pl.BlockSpec((B, NCLS), lambda: (0, 0)),
        out_shape=jax.ShapeDtypeStruct((B, NCLS), jnp.float32),
    )(*gargs)
    return out


# speedup vs baseline: 1.0998x; 1.0660x over previous
import functools

import jax
import jax.numpy as jnp
from jax import lax
from jax.experimental import pallas as pl
from jax.experimental.pallas import tpu as pltpu
from jax.experimental.pallas import tpu_sc as plsc

B = 128
N = 512
F = 64
PE = 500
GE = 5000
HID = 128
GED = 128
CLIN = 6
NCLS = 2

BP = 8
NWORK = 32
PPW = B // NWORK
NWRD = N // 16
BMP = N * NWRD
PEP = 512
GEP = 5120


GSH = B * B // 16
GEW = GEP // 16


def _sc_build_body(pe_hbm, ges_hbm, ged_hbm, bm_hbm, outg_hbm,
                   zbuf, wbuf, ebuf, gsrc, gdst, ibuf, vbuf, sem):
    cid = lax.axis_index("c")
    sid = lax.axis_index("s")
    wid = sid * 2 + cid
    lane = lax.broadcasted_iota(jnp.int32, (16,), 0)

    @pl.when(cid == 0)
    def _():
        def _fill_z(i, _):
            zbuf[pl.ds(i * 16, 16)] = jnp.zeros((16,), jnp.float32)
            return 0

        def _fill_o(i, _):
            vbuf[pl.ds(i * 16, 16)] = jnp.ones((16,), jnp.float32)
            return 0

        lax.fori_loop(0, GSH // 16, _fill_z, 0, unroll=8)
        lax.fori_loop(0, GEW // 16, _fill_o, 0, unroll=8)
        pltpu.sync_copy(zbuf, outg_hbm.at[pl.ds(sid * GSH, GSH)])
        plsc.subcore_barrier()
        pltpu.sync_copy(ges_hbm.at[pl.ds(sid * GEW, GEW)], gsrc)
        pltpu.sync_copy(ged_hbm.at[pl.ds(sid * GEW, GEW)], gdst)

        def _gidx(c, _):
            s = gsrc[pl.ds(c * 16, 16)]
            d = gdst[pl.ds(c * 16, 16)]
            ibuf[pl.ds(c * 16, 16)] = d * B + s
            return 0

        lax.fori_loop(0, GEW // 16, _gidx, 0, unroll=4)
        pltpu.async_copy(vbuf, outg_hbm.at[ibuf], sem).wait()

    for j in range(PPW):
        b = wid * PPW + j

        def _zero_w(i, _):
            wbuf[pl.ds(i * 16, 16)] = jnp.zeros((16,), jnp.int32)
            return 0

        lax.fori_loop(0, BMP // 16, _zero_w, 0, unroll=8)
        pltpu.sync_copy(pe_hbm.at[b], ebuf)

        def _edges(c, _):
            s = ebuf[0, pl.ds(c * 16, 16)]
            d = ebuf[1, pl.ds(c * 16, 16)]
            widx = d * NWRD + lax.shift_right_logical(s, 4)
            bit = lax.shift_left(jnp.full((16,), 1, jnp.int32),
                                 lax.bitwise_and(s, 15))
            mask = (c * 16 + lane) < PE
            plsc.addupdate_scatter(wbuf, [widx], bit, mask=mask)
            return 0

        lax.fori_loop(0, PEP // 16, _edges, 0, unroll=4)
        pltpu.sync_copy(wbuf, bm_hbm.at[pl.ds(b * BMP, BMP)])


@functools.partial(
    pl.kernel,
    out_type=(jax.ShapeDtypeStruct((B * BMP,), jnp.int32),
              jax.ShapeDtypeStruct((B * B,), jnp.float32)),
    mesh=plsc.VectorSubcoreMesh(core_axis_name="c", subcore_axis_name="s"),
    scratch_types=[
        pltpu.VMEM((GSH,), jnp.float32),
        pltpu.VMEM((BMP,), jnp.int32),
        pltpu.VMEM((2, PEP), jnp.int32),
        pltpu.VMEM((GEW,), jnp.int32),
        pltpu.VMEM((GEW,), jnp.int32),
        pltpu.VMEM((GEW,), jnp.int32),
        pltpu.VMEM((GEW,), jnp.float32),
        pltpu.SemaphoreType.DMA,
    ],
    compiler_params=pltpu.CompilerParams(needs_layout_passes=False),
)
def _sc_build(pe_hbm, ges_hbm, ged_hbm, bm_hbm, outg_hbm,
              zbuf, wbuf, ebuf, gsrc, gdst, ibuf, vbuf, sem):
    _sc_build_body(pe_hbm, ges_hbm, ged_hbm, bm_hbm, outg_hbm,
                   zbuf, wbuf, ebuf, gsrc, gdst, ibuf, vbuf, sem)


def _patient_body(a_ref, x_ref, w1_ref, b1_ref, w2_ref, b2_ref,
                  w3_ref, b3_ref, plw_ref, plb_ref, out_ref):
    w1 = w1_ref[...]
    w2 = w2_ref[...]
    w3 = w3_ref[...]
    b1 = b1_ref[...]
    b2 = b2_ref[...]
    b3 = b3_ref[...]
    sel = (lax.broadcasted_iota(jnp.int32, (NWRD, N), 1) // 16
           == lax.broadcasted_iota(jnp.int32, (NWRD, N), 0)
           ).astype(jnp.float32)
    shift = lax.broadcasted_iota(jnp.int32, (N, N), 1) & 15

    def _expand(p):
        bmf = a_ref[p].astype(jnp.float32)
        v = jnp.dot(bmf, sel, preferred_element_type=jnp.float32)
        return ((v.astype(jnp.int32) >> shift) & 1).astype(jnp.float32)

    nxt = _expand(0)
    for p in range(BP):
        abar = nxt
        if p + 1 < BP:
            nxt = _expand(p + 1)
        deg = jnp.sum(abar, axis=1, keepdims=True) + 1.0
        dis = lax.rsqrt(deg)
        dis2 = dis * dis
        h = x_ref[p]
        for w, bb in ((w1, b1), (w2, b2), (w3, b3)):
            z = jnp.dot(h, w, preferred_element_type=jnp.float32)
            zn = dis * z
            acc = jnp.dot(abar, zn, preferred_element_type=jnp.float32)
            h = jnp.maximum(dis * acc + dis2 * z + bb, 0.0)
        g = jnp.max(h, axis=0, keepdims=True)
        out_ref[p:p + 1, :] = (
            jnp.dot(g, plw_ref[...], preferred_element_type=jnp.float32)
            + plb_ref[...])


def _group_body(ag_ref, emb_ref, demo_ref, w1a_ref, w1b_ref, b1_ref,
                w2_ref, b2_ref, w3_ref, b3_ref, w4_ref, b4_ref, out_ref):
    abar = ag_ref[...]
    deg = jnp.sum(abar, axis=1, keepdims=True) + 1.0
    dis = lax.rsqrt(deg)
    dis2 = dis * dis
    an = dis * abar * jnp.transpose(dis)

    z = (jnp.dot(emb_ref[...], w1a_ref[...], preferred_element_type=jnp.float32)
         + jnp.dot(demo_ref[...], w1b_ref[...], preferred_element_type=jnp.float32))
    h = jnp.maximum(jnp.dot(an, z, preferred_element_type=jnp.float32)
                    + dis2 * z + b1_ref[...], 0.0)
    for w_ref, b_ref, act in ((w2_ref, b2_ref, True), (w3_ref, b3_ref, True),
                              (w4_ref, b4_ref, False)):
        z = jnp.dot(h, w_ref[...], preferred_element_type=jnp.float32)
        h = (jnp.dot(an, z, preferred_element_type=jnp.float32)
             + dis2 * z + b_ref[...])
        if act:
            h = jnp.maximum(h, 0.0)
    m = jnp.max(h, axis=1, keepdims=True)
    y = h - m
    out_ref[...] = y - jnp.log(jnp.sum(jnp.exp(y), axis=1, keepdims=True))


def kernel(x, demographic, patient_edge_idx, group_edge_idx,
           pW1, pb1, pW2, pb2, pW3, pb3, plinW, plinb,
           gW1, gb1, gW2, gb2, gW3, gb3, gW4, gb4):
    pe_pad = jnp.concatenate(
        [patient_edge_idx,
         jnp.tile(patient_edge_idx[:, :, -1:], (1, 1, PEP - PE))], axis=2)
    ge_pad = jnp.concatenate(
        [group_edge_idx,
         jnp.tile(group_edge_idx[:, -1:], (1, GEP - GE))], axis=1)

    bm_flat, ag_flat = _sc_build(pe_pad, ge_pad[0], ge_pad[1])
    bm = bm_flat.reshape(B, N, NWRD)
    ag = ag_flat.reshape(B, B)

    row = lambda v: v.reshape(1, -1)
    fullg = lambda a: pl.BlockSpec(a.shape, lambda i: (0,) * a.ndim)
    full = lambda a: pl.BlockSpec(a.shape, lambda: (0,) * a.ndim)

    wargs = (pW1, row(pb1), pW2, row(pb2), pW3, row(pb3), plinW, row(plinb))
    embed = pl.pallas_call(
        _patient_body,
        grid=(B // BP,),
        in_specs=[
            pl.BlockSpec((BP, N, NWRD), lambda i: (i, 0, 0)),
            pl.BlockSpec((BP, N, F), lambda i: (i, 0, 0)),
        ] + [fullg(a) for a in wargs],
        out_specs=pl.BlockSpec((BP, GED), lambda i: (i, 0)),
        out_shape=jax.ShapeDtypeStruct((B, GED), jnp.float32),
    )(bm, x, *wargs)

    gw1a = gW1[:GED]
    gw1b = gW1[GED:]
    gargs = (ag, embed, demographic, gw1a, gw1b, row(gb1),
             gW2, row(gb2), gW3, row(gb3), gW4, row(gb4))
    out = pl.pallas_call(
        _group_body,
        in_specs=[full(a) for a in gargs],
        out_specs=pl.BlockSpec((B, NCLS), lambda: (0, 0)),
        out_shape=jax.ShapeDtypeStruct((B, NCLS), jnp.float32),
    )(*gargs)
    return out
